# baseline (speedup 1.0000x reference)
import functools

import jax
import jax.numpy as jnp
from jax.experimental import pallas as pl
from jax.experimental.pallas import tpu as pltpu

_NUM_EXPERTS = 64
_TOP_K = 8
_HIDDEN = 2048
_BLOCK_T = 512
_NUM_BLOCKS = 32


def _topk_from(p, w_out_ref, i_out_ref):
    lanef = jax.lax.broadcasted_iota(jnp.int32, p.shape, 1).astype(jnp.float32)
    vals = []
    idxfs = []
    neg = jnp.float32(-jnp.inf)
    big = jnp.float32(_NUM_EXPERTS)
    for _ in range(_TOP_K):
        top = jnp.max(p, axis=1, keepdims=True)
        idxf = jnp.min(jnp.where(p == top, lanef, big), axis=1, keepdims=True)
        vals.append(top)
        idxfs.append(idxf)
        p = jnp.where(lanef == idxf, neg, p)
    topk = jnp.concatenate(vals, axis=1)
    e = jnp.exp(topk - vals[0])
    w_out_ref[...] = e / jnp.sum(e, axis=1, keepdims=True)
    i_out_ref[...] = jnp.concatenate(idxfs, axis=1).astype(jnp.int32)


def _router_block(x_ref, wt_ref, w_out_ref, i_out_ref, s_ref):
    p = s_ref[...]
    _topk_from(p, w_out_ref, i_out_ref)
    s_ref[...] = jnp.dot(
        x_ref[...], wt_ref[...], preferred_element_type=jnp.float32
    )


@functools.partial(jax.jit, static_argnames=())
def kernel(hidden_states, W):
    b, s, h = hidden_states.shape
    n = b * s
    x = hidden_states.reshape(n, h)
    wt = W.T
    last = _NUM_BLOCKS - 1
    weights, indices = pl.pallas_call(
        _router_block,
        grid=(_NUM_BLOCKS + 1,),
        in_specs=[
            pl.BlockSpec((_BLOCK_T, h), lambda i: (jnp.minimum(i, last), 0)),
            pl.BlockSpec((h, _NUM_EXPERTS), lambda i: (0, 0)),
        ],
        out_specs=[
            pl.BlockSpec((_BLOCK_T, _TOP_K), lambda i: (jnp.maximum(i - 1, 0), 0)),
            pl.BlockSpec((_BLOCK_T, _TOP_K), lambda i: (jnp.maximum(i - 1, 0), 0)),
        ],
        out_shape=[
            jax.ShapeDtypeStruct((n, _TOP_K), jnp.float32),
            jax.ShapeDtypeStruct((n, _TOP_K), jnp.int32),
        ],
        scratch_shapes=[
            pltpu.VMEM((_BLOCK_T, _NUM_EXPERTS), jnp.float32),
        ],
    )(x, wt)
    return (weights.reshape(b, s, _TOP_K), indices.reshape(b, s, _TOP_K))


# speedup vs baseline: 1.1461x; 1.1461x over previous
import functools

import jax
import jax.numpy as jnp
from jax.experimental import pallas as pl
from jax.experimental.pallas import tpu as pltpu

_NUM_EXPERTS = 64
_TOP_K = 8
_HIDDEN = 2048
_BLOCK_T = 1024
_NUM_BLOCKS = 16


def _topk_from(p, w_out_ref, i_out_ref):
    lanef = jax.lax.broadcasted_iota(jnp.int32, p.shape, 1).astype(jnp.float32)
    vals = []
    idxfs = []
    neg = jnp.float32(-jnp.inf)
    big = jnp.float32(_NUM_EXPERTS)
    for _ in range(_TOP_K):
        top = jnp.max(p, axis=1, keepdims=True)
        idxf = jnp.min(jnp.where(p == top, lanef, big), axis=1, keepdims=True)
        vals.append(top)
        idxfs.append(idxf)
        p = jnp.where(lanef == idxf, neg, p)
    topk = jnp.concatenate(vals, axis=1)
    e = jnp.exp(topk - vals[0])
    w_out_ref[...] = e / jnp.sum(e, axis=1, keepdims=True)
    i_out_ref[...] = jnp.concatenate(idxfs, axis=1).astype(jnp.int32)


def _router_block(x_ref, wt_ref, w_out_ref, i_out_ref, s_ref):
    p = s_ref[...]
    _topk_from(p, w_out_ref, i_out_ref)
    s_ref[...] = jnp.dot(
        x_ref[...], wt_ref[...], preferred_element_type=jnp.float32
    )


@functools.partial(jax.jit, static_argnames=())
def kernel(hidden_states, W):
    b, s, h = hidden_states.shape
    n = b * s
    x = hidden_states.reshape(n, h)
    wt = W.T
    last = _NUM_BLOCKS - 1
    weights, indices = pl.pallas_call(
        _router_block,
        grid=(_NUM_BLOCKS + 1,),
        in_specs=[
            pl.BlockSpec((_BLOCK_T, h), lambda i: (jnp.minimum(i, last), 0)),
            pl.BlockSpec((h, _NUM_EXPERTS), lambda i: (0, 0)),
        ],
        out_specs=[
            pl.BlockSpec((_BLOCK_T, _TOP_K), lambda i: (jnp.maximum(i - 1, 0), 0)),
            pl.BlockSpec((_BLOCK_T, _TOP_K), lambda i: (jnp.maximum(i - 1, 0), 0)),
        ],
        out_shape=[
            jax.ShapeDtypeStruct((n, _TOP_K), jnp.float32),
            jax.ShapeDtypeStruct((n, _TOP_K), jnp.int32),
        ],
        scratch_shapes=[
            pltpu.VMEM((_BLOCK_T, _NUM_EXPERTS), jnp.float32),
        ],
    )(x, wt)
    return (weights.reshape(b, s, _TOP_K), indices.reshape(b, s, _TOP_K))


# speedup vs baseline: 1.1895x; 1.0378x over previous
import functools

import jax
import jax.numpy as jnp
from jax.experimental import pallas as pl
from jax.experimental.pallas import tpu as pltpu

_NUM_EXPERTS = 64
_TOP_K = 8
_HIDDEN = 2048
_BLOCK_T = 1024
_NUM_BLOCKS = 16


def _topk_from(p, w_out_ref, i_out_ref):
    lanef = jax.lax.broadcasted_iota(jnp.int32, p.shape, 1).astype(jnp.float32)
    vals = []
    idxfs = []
    neg = jnp.float32(-jnp.inf)
    big = jnp.float32(_NUM_EXPERTS)
    for _ in range(_TOP_K):
        top = jnp.max(p, axis=1, keepdims=True)
        idxf = jnp.min(jnp.where(p == top, lanef, big), axis=1, keepdims=True)
        vals.append(top)
        idxfs.append(idxf)
        p = jnp.where(lanef == idxf, neg, p)
    topk = jnp.concatenate(vals, axis=1)
    e = jnp.exp(topk - vals[0])
    w_out_ref[...] = e / jnp.sum(e, axis=1, keepdims=True)
    i_out_ref[...] = jnp.concatenate(idxfs, axis=1).astype(jnp.int32)


def _router_block(x_ref, wt_ref, w_out_ref, i_out_ref, s_ref):
    p = s_ref[...]
    _topk_from(p, w_out_ref, i_out_ref)
    s_ref[...] = jax.lax.dot_general(
        x_ref[...],
        wt_ref[...],
        (((1,), (1,)), ((), ())),
        preferred_element_type=jnp.float32,
    )


@functools.partial(jax.jit, static_argnames=())
def kernel(hidden_states, W):
    b, s, h = hidden_states.shape
    n = b * s
    x = hidden_states.reshape(n, h)
    last = _NUM_BLOCKS - 1
    weights, indices = pl.pallas_call(
        _router_block,
        grid=(_NUM_BLOCKS + 1,),
        in_specs=[
            pl.BlockSpec((_BLOCK_T, h), lambda i: (jnp.minimum(i, last), 0)),
            pl.BlockSpec((_NUM_EXPERTS, h), lambda i: (0, 0)),
        ],
        out_specs=[
            pl.BlockSpec((_BLOCK_T, _TOP_K), lambda i: (jnp.maximum(i - 1, 0), 0)),
            pl.BlockSpec((_BLOCK_T, _TOP_K), lambda i: (jnp.maximum(i - 1, 0), 0)),
        ],
        out_shape=[
            jax.ShapeDtypeStruct((n, _TOP_K), jnp.float32),
            jax.ShapeDtypeStruct((n, _TOP_K), jnp.int32),
        ],
        scratch_shapes=[
            pltpu.VMEM((_BLOCK_T, _NUM_EXPERTS), jnp.float32),
        ],
    )(x, W)
    return (weights.reshape(b, s, _TOP_K), indices.reshape(b, s, _TOP_K))
